# baseline (speedup 1.0000x reference)
import jax
import jax.numpy as jnp
from jax.experimental import pallas as pl
from jax.experimental.pallas import tpu as pltpu

N_X = 4096
N_T = 4096
LANES = 128
SUB = N_T // LANES
R_TOTAL = N_X * SUB
BLK_X = 512
BLK_R = BLK_X * SUB


def _bitonic_sort_2d(a):
    r_iota = jax.lax.broadcasted_iota(jnp.int32, (SUB, LANES), 0)
    c_iota = jax.lax.broadcasted_iota(jnp.int32, (SUB, LANES), 1)
    idx = r_iota * LANES + c_iota
    n = SUB * LANES
    k = 2
    while k <= n:
        j = k // 2
        while j >= 1:
            if j < LANES:
                fwd = jnp.roll(a, -j, axis=1)
                bwd = jnp.roll(a, j, axis=1)
            else:
                jr = j // LANES
                fwd = jnp.roll(a, -jr, axis=0)
                bwd = jnp.roll(a, jr, axis=0)
            lower = (idx & j) == 0
            p = jnp.where(lower, fwd, bwd)
            asc = (idx & k) == 0
            keep_min = lower == asc
            a = jnp.where(keep_min, jnp.minimum(a, p), jnp.maximum(a, p))
            j //= 2
        k *= 2
    return a


def _fused_kernel(x_col, t2d, xx_ref, tt_ref, ts2d):
    i = pl.program_id(0)

    @pl.when(i == 0)
    def _sort():
        ts2d[:] = _bitonic_sort_2d(t2d[:])

    xb = x_col[:].reshape(BLK_X, 1, 1)
    xx_ref[:] = jnp.broadcast_to(xb, (BLK_X, SUB, LANES)).reshape(BLK_R, LANES)
    ts = ts2d[:]
    tt_ref[:] = jnp.broadcast_to(ts[None], (BLK_X, SUB, LANES)).reshape(BLK_R, LANES)


@jax.jit
def kernel(x_grid, t_grid):
    x_col = x_grid.reshape(N_X, 1)
    t2d = t_grid.reshape(SUB, LANES)
    xx, tt = pl.pallas_call(
        _fused_kernel,
        grid=(N_X // BLK_X,),
        in_specs=[
            pl.BlockSpec((BLK_X, 1), lambda i: (i, 0)),
            pl.BlockSpec((SUB, LANES), lambda i: (0, 0)),
        ],
        out_specs=[
            pl.BlockSpec((BLK_R, LANES), lambda i: (i, 0)),
            pl.BlockSpec((BLK_R, LANES), lambda i: (i, 0)),
        ],
        out_shape=[
            jax.ShapeDtypeStruct((R_TOTAL, LANES), jnp.float32),
            jax.ShapeDtypeStruct((R_TOTAL, LANES), jnp.float32),
        ],
        scratch_shapes=[
            pltpu.VMEM((SUB, LANES), jnp.float32),
        ],
        compiler_params=pltpu.CompilerParams(
            vmem_limit_bytes=110 * 1024 * 1024,
        ),
    )(x_col, t2d)
    return (xx.reshape(-1, 1), tt.reshape(-1, 1))


# speedup vs baseline: 1.0634x; 1.0634x over previous
import jax
import jax.numpy as jnp
from jax.experimental import pallas as pl
from jax.experimental.pallas import tpu as pltpu

N_X = 4096
N_T = 4096
LANES = 128
SUB = N_T // LANES
R_TOTAL = N_X * SUB
BLK_X = 256
BLK_R = BLK_X * SUB


def _bitonic_sort_2d(a):
    r_iota = jax.lax.broadcasted_iota(jnp.int32, (SUB, LANES), 0)
    c_iota = jax.lax.broadcasted_iota(jnp.int32, (SUB, LANES), 1)
    idx = r_iota * LANES + c_iota
    n = SUB * LANES
    k = 2
    while k <= n:
        j = k // 2
        while j >= 1:
            if j < LANES:
                fwd = jnp.roll(a, -j, axis=1)
                bwd = jnp.roll(a, j, axis=1)
            else:
                jr = j // LANES
                fwd = jnp.roll(a, -jr, axis=0)
                bwd = jnp.roll(a, jr, axis=0)
            lower = (idx & j) == 0
            p = jnp.where(lower, fwd, bwd)
            asc = (idx & k) == 0
            keep_min = lower == asc
            a = jnp.where(keep_min, jnp.minimum(a, p), jnp.maximum(a, p))
            j //= 2
        k *= 2
    return a


def _fused_kernel(x_col, t2d, xx_ref, tt_ref, ts2d):
    i = pl.program_id(0)

    @pl.when(i == 0)
    def _sort():
        ts2d[:] = _bitonic_sort_2d(t2d[:])

    xb = x_col[:].reshape(BLK_X, 1, 1)
    xx_ref[:] = jnp.broadcast_to(xb, (BLK_X, SUB, LANES)).reshape(BLK_R, LANES)
    ts = ts2d[:]
    tt_ref[:] = jnp.broadcast_to(ts[None], (BLK_X, SUB, LANES)).reshape(BLK_R, LANES)


@jax.jit
def kernel(x_grid, t_grid):
    x_col = x_grid.reshape(N_X, 1)
    t2d = t_grid.reshape(SUB, LANES)
    xx, tt = pl.pallas_call(
        _fused_kernel,
        grid=(N_X // BLK_X,),
        in_specs=[
            pl.BlockSpec((BLK_X, 1), lambda i: (i, 0)),
            pl.BlockSpec((SUB, LANES), lambda i: (0, 0)),
        ],
        out_specs=[
            pl.BlockSpec((BLK_R, LANES), lambda i: (i, 0)),
            pl.BlockSpec((BLK_R, LANES), lambda i: (i, 0)),
        ],
        out_shape=[
            jax.ShapeDtypeStruct((R_TOTAL, LANES), jnp.float32),
            jax.ShapeDtypeStruct((R_TOTAL, LANES), jnp.float32),
        ],
        scratch_shapes=[
            pltpu.VMEM((SUB, LANES), jnp.float32),
        ],
    )(x_col, t2d)
    return (xx.reshape(-1, 1), tt.reshape(-1, 1))
